# baseline (speedup 1.0000x reference)
import functools

import jax
import jax.numpy as jnp
from jax import lax
from jax.experimental import pallas as pl
from jax.experimental.pallas import tpu as pltpu
from jax.experimental.pallas import tpu_sc as plsc

_NUM_CORES = 2
_NUM_SUBCORES = 16
_LANES = 16
_ROWS_PER_DMA = 8


@functools.partial(jax.jit, static_argnums=(1, 2))
def _one_hot_sc(x_flat, d, j):
    bj = x_flat.shape[0]
    b = bj // j
    nw = _NUM_CORES * _NUM_SUBCORES
    b_per_w = b // nw
    rpd = _ROWS_PER_DMA
    assert b % nw == 0 and b_per_w % rpd == 0
    dj = d * j
    assert dj % _LANES == 0 and j >= _LANES
    ndma = b_per_w // rpd
    starts = list(range(0, j - _LANES + 1, _LANES))
    if j % _LANES:
        starts.append(j - _LANES)
    ncs = len(starts)
    epr = ncs * _LANES
    assert 128 % epr == 0 or epr % 128 == 0
    rows_per_iblock = max(1, 128 // epr)
    niblocks = b_per_w * epr // 128

    mesh = plsc.VectorSubcoreMesh(
        core_axis_name="c", subcore_axis_name="s",
        num_cores=_NUM_CORES, num_subcores=_NUM_SUBCORES)

    @functools.partial(
        pl.kernel,
        mesh=mesh,
        compiler_params=pltpu.CompilerParams(
            needs_layout_passes=False, use_tc_tiling_on_sc=True),
        out_type=jax.ShapeDtypeStruct((b * dj,), jnp.float32),
        scratch_types=[
            pltpu.VMEM((b_per_w * j,), jnp.int32),
            pltpu.VMEM((niblocks, 128), jnp.int32),
            pltpu.VMEM((128,), jnp.float32),
            pltpu.VMEM((dj,), jnp.float32),
            pltpu.VMEM_SHARED((rpd * dj,), jnp.float32),
            pltpu.SemaphoreType.DMA,
            pltpu.SemaphoreType.DMA,
        ],
    )
    def run(x_hbm, out_hbm, xv, idxv, onev, zslab, zshared, sem_z, sem_s):
        cid = lax.axis_index("c")
        sid = lax.axis_index("s")
        wid = sid * _NUM_CORES + cid
        base = wid * b_per_w

        pltpu.sync_copy(x_hbm.at[pl.ds(base * j, b_per_w * j)], xv)

        zf = jnp.zeros((_LANES,), jnp.float32)
        onef = jnp.full((_LANES,), 1.0, jnp.float32)
        lane = lax.iota(jnp.int32, _LANES)

        for c in range(128 // _LANES):
            onev[pl.ds(c * _LANES, _LANES)] = onef

        def idx_body(i, carry):
            for half in range(rows_per_iblock):
                row = i * rows_per_iblock + half
                for ci, s in enumerate(starts):
                    xchunk = xv[pl.ds(row * j + s, _LANES)]
                    pos = (base + row) * dj + xchunk * j + (lane + s)
                    col = half * epr + ci * _LANES
                    idxv[i, pl.ds(col, _LANES)] = pos
            return carry

        lax.fori_loop(0, niblocks, idx_body, 0)

        def zslab_body(i, carry):
            zslab[pl.ds(i * _LANES, _LANES)] = zf
            return carry

        lax.fori_loop(0, dj // _LANES, zslab_body, 0)

        @pl.when(sid == 0)
        def _fill_shared():
            for r in range(rpd):
                pltpu.sync_copy(zslab, zshared.at[pl.ds(r * dj, dj)])

        plsc.subcore_barrier()

        for i in range(ndma):
            off = (base + i * rpd) * dj
            pltpu.async_copy(zshared, out_hbm.at[pl.ds(off, rpd * dj)], sem_z)
        for i in range(ndma):
            off = (base + i * rpd) * dj
            pltpu.make_async_copy(
                zshared, out_hbm.at[pl.ds(off, rpd * dj)], sem_z).wait()

        def fire_body(i, carry):
            pltpu.async_copy(onev, out_hbm.at[idxv.at[i]], sem_s)
            return carry

        lax.fori_loop(0, niblocks, fire_body, 0)

        def drain_body(i, carry):
            pltpu.make_async_copy(onev, out_hbm.at[idxv.at[i]], sem_s).wait()
            return carry

        lax.fori_loop(0, niblocks, drain_body, 0)

    return run(x_flat)


def kernel(X_in, ones):
    b, j = X_in.shape
    d = ones.shape[0]
    out = _one_hot_sc(X_in.reshape(-1), d, j)
    return out.reshape(b, d, j)


# speedup vs baseline: 15.0712x; 15.0712x over previous
import functools

import jax
import jax.numpy as jnp
from jax import lax
from jax.experimental import pallas as pl
from jax.experimental.pallas import tpu as pltpu
from jax.experimental.pallas import tpu_sc as plsc

_NUM_CORES = 2
_NUM_SUBCORES = 16
_LANES = 16
_DBLK = 256


@functools.partial(jax.jit, static_argnums=(1,))
def _one_hot_sc(x, d):
    b, j = x.shape
    nw = _NUM_CORES * _NUM_SUBCORES
    bw = b // nw
    assert b % nw == 0 and bw == 128
    nchunk = bw // _LANES
    dblocks = []
    d0 = 0
    while d0 < d:
        dblocks.append((d0, min(_DBLK, d - d0)))
        d0 += _DBLK
    nq = len(dblocks)
    assert nq % 2 == 0

    mesh = plsc.VectorSubcoreMesh(
        core_axis_name="c", subcore_axis_name="s",
        num_cores=_NUM_CORES, num_subcores=_NUM_SUBCORES)

    @functools.partial(
        pl.kernel,
        mesh=mesh,
        compiler_params=pltpu.CompilerParams(needs_layout_passes=False),
        out_type=jax.ShapeDtypeStruct((j, d, b), jnp.float32),
        scratch_types=[
            pltpu.VMEM((bw, j), jnp.int32),
            pltpu.VMEM((_DBLK, bw), jnp.float32),
            pltpu.VMEM((_DBLK, bw), jnp.float32),
            pltpu.SemaphoreType.DMA,
            pltpu.SemaphoreType.DMA,
        ],
    )
    def run(x_hbm, out_hbm, xv, slab0, slab1, sem0, sem1):
        cid = lax.axis_index("c")
        sid = lax.axis_index("s")
        wid = sid * _NUM_CORES + cid
        b0 = wid * bw

        pltpu.sync_copy(x_hbm.at[pl.ds(b0, bw)], xv)

        zf = jnp.zeros((_LANES,), jnp.float32)
        onef = jnp.full((_LANES,), 1.0, jnp.float32)
        lane = lax.iota(jnp.int32, _LANES)

        def zero_body(i, carry):
            r = i // (bw // _LANES)
            c = (i % (bw // _LANES)) * _LANES
            slab0[r, pl.ds(c, _LANES)] = zf
            slab1[r, pl.ds(c, _LANES)] = zf
            return carry

        lax.fori_loop(0, _DBLK * bw // _LANES, zero_body, 0)

        slabs = (slab0, slab1)
        sems = (sem0, sem1)

        def xcol(jj):
            cols = jnp.full((_LANES,), jj, jnp.int32)
            return [plsc.load_gather(xv, [lane + c * _LANES, cols])
                    for c in range(nchunk)]

        def scatter(slab, xs, dlo, dsz, val):
            for c in range(nchunk):
                xc = xs[c]
                row = xc - dlo
                mask = (xc >= dlo) & (xc < dlo + dsz)
                plsc.store_scatter(slab, [row, lane + c * _LANES],
                                   val, mask=mask)

        def dma(slab, sem, jj, dlo, dsz):
            return pltpu.async_copy(
                slab.at[pl.ds(0, dsz)],
                out_hbm.at[jj, pl.ds(dlo, dsz), pl.ds(b0, bw)], sem)

        def drain(slab, sem, jj, dlo, dsz):
            pltpu.make_async_copy(
                slab.at[pl.ds(0, dsz)],
                out_hbm.at[jj, pl.ds(dlo, dsz), pl.ds(b0, bw)], sem).wait()

        def unit(jj, q, first):
            dlo, dsz = dblocks[q]
            slab, sem = slabs[q % 2], sems[q % 2]
            qp = (q + nq - 2) % nq
            dlop, dszp = dblocks[qp]
            jjp = jj - (1 if q < 2 else 0)
            if not first:
                drain(slab, sem, jjp, dlop, dszp)
                scatter(slab, xcol(jjp), dlop, dszp, zf)
            xs = xcol(jj)
            scatter(slab, xs, dlo, dsz, onef)
            dma(slab, sem, jj, dlo, dsz)

        unit(0, 0, True)
        unit(0, 1, True)

        def jj_body(jj, carry):
            for q in range(2, nq):
                unit(jj, q, False)
            for q in range(2):
                unit(jj + 1, q, False)
            return carry

        lax.fori_loop(0, j - 1, jj_body, 0)
        for q in range(2, nq):
            unit(j - 1, q, False)

        for q in (nq - 2, nq - 1):
            dlo, dsz = dblocks[q]
            drain(slabs[q % 2], sems[q % 2], j - 1, dlo, dsz)

    return run(x)


def kernel(X_in, ones):
    d = ones.shape[0]
    out = _one_hot_sc(X_in, d)
    return jnp.transpose(out, (2, 1, 0))


# speedup vs baseline: 15.5947x; 1.0347x over previous
import functools

import jax
import jax.numpy as jnp
from jax import lax
from jax.experimental import pallas as pl
from jax.experimental.pallas import tpu as pltpu
from jax.experimental.pallas import tpu_sc as plsc

_NUM_CORES = 2
_NUM_SUBCORES = 16
_LANES = 16
_DBLK = 256


@functools.partial(jax.jit, static_argnums=(1,))
def _one_hot_sc(x, d):
    j, b = x.shape
    nw = _NUM_CORES * _NUM_SUBCORES
    bw = b // nw
    assert b % nw == 0 and bw == 128
    nchunk = bw // _LANES
    dblocks = []
    d0 = 0
    while d0 < d:
        dblocks.append((d0, min(_DBLK, d - d0)))
        d0 += _DBLK
    nq = len(dblocks)
    assert nq % 2 == 0

    mesh = plsc.VectorSubcoreMesh(
        core_axis_name="c", subcore_axis_name="s",
        num_cores=_NUM_CORES, num_subcores=_NUM_SUBCORES)

    @functools.partial(
        pl.kernel,
        mesh=mesh,
        compiler_params=pltpu.CompilerParams(needs_layout_passes=False),
        out_type=jax.ShapeDtypeStruct((j, d, b), jnp.float32),
        scratch_types=[
            pltpu.VMEM((j, bw), jnp.int32),
            pltpu.VMEM((_DBLK, bw), jnp.float32),
            pltpu.VMEM((_DBLK, bw), jnp.float32),
            pltpu.SemaphoreType.DMA,
            pltpu.SemaphoreType.DMA,
        ],
    )
    def run(x_hbm, out_hbm, xv, slab0, slab1, sem0, sem1):
        cid = lax.axis_index("c")
        sid = lax.axis_index("s")
        wid = sid * _NUM_CORES + cid
        b0 = wid * bw

        pltpu.sync_copy(x_hbm.at[pl.ds(0, j), pl.ds(b0, bw)], xv)

        zf = jnp.zeros((_LANES,), jnp.float32)
        onef = jnp.full((_LANES,), 1.0, jnp.float32)
        lane = lax.iota(jnp.int32, _LANES)

        def zero_body(i, carry):
            r = i // (bw // _LANES)
            c = (i % (bw // _LANES)) * _LANES
            slab0[r, pl.ds(c, _LANES)] = zf
            slab1[r, pl.ds(c, _LANES)] = zf
            return carry

        lax.fori_loop(0, _DBLK * bw // _LANES, zero_body, 0)

        slabs = (slab0, slab1)
        sems = (sem0, sem1)

        def xcol(jj):
            cols = jnp.full((_LANES,), jj, jnp.int32)
            return [plsc.load_gather(xv, [cols, lane + c * _LANES])
                    for c in range(nchunk)]

        def scatter(slab, xs, dlo, dsz, val):
            for c in range(nchunk):
                xc = xs[c]
                row = xc - dlo
                mask = (xc >= dlo) & (xc < dlo + dsz)
                plsc.store_scatter(slab, [row, lane + c * _LANES],
                                   val, mask=mask)

        def dma(slab, sem, jj, dlo, dsz):
            return pltpu.async_copy(
                slab.at[pl.ds(0, dsz)],
                out_hbm.at[jj, pl.ds(dlo, dsz), pl.ds(b0, bw)], sem)

        def drain(slab, sem, jj, dlo, dsz):
            pltpu.make_async_copy(
                slab.at[pl.ds(0, dsz)],
                out_hbm.at[jj, pl.ds(dlo, dsz), pl.ds(b0, bw)], sem).wait()

        def unit(jj, q, first):
            dlo, dsz = dblocks[q]
            slab, sem = slabs[q % 2], sems[q % 2]
            qp = (q + nq - 2) % nq
            dlop, dszp = dblocks[qp]
            jjp = jj - (1 if q < 2 else 0)
            if not first:
                drain(slab, sem, jjp, dlop, dszp)
                scatter(slab, xcol(jjp), dlop, dszp, zf)
            xs = xcol(jj)
            scatter(slab, xs, dlo, dsz, onef)
            dma(slab, sem, jj, dlo, dsz)

        unit(0, 0, True)
        unit(0, 1, True)

        def jj_body(jj, carry):
            for q in range(2, nq):
                unit(jj, q, False)
            for q in range(2):
                unit(jj + 1, q, False)
            return carry

        lax.fori_loop(0, j - 1, jj_body, 0)
        for q in range(2, nq):
            unit(j - 1, q, False)

        for q in (nq - 2, nq - 1):
            dlo, dsz = dblocks[q]
            drain(slabs[q % 2], sems[q % 2], j - 1, dlo, dsz)

    return run(x)


def kernel(X_in, ones):
    d = ones.shape[0]
    out = _one_hot_sc(X_in.T, d)
    return jnp.transpose(out, (2, 1, 0))
